# speedup vs baseline: 1.5478x; 1.5478x over previous
import jax
import jax.numpy as jnp
from jax import lax
from jax.experimental import pallas as pl
from jax.experimental.pallas import tpu as pltpu
from jax.experimental.pallas import tpu_sc as plsc

VMIN, VMAX, BINS, WIDTH = 0.0, 1.0, 32, 128

NC, NS, L = 2, 16, 16
NW = NC * NS
TOTAL = 16384 * 50
PER_W = TOTAL // NW
CHUNK = 512
NSUB = 4
SUB = CHUNK // NSUB
NCHUNK = PER_W // CHUNK


def _body(x_hbm, table_hbm, out_hbm, xv, idxv, rows, sem):
    wid = lax.axis_index("s") * NC + lax.axis_index("c")
    base = wid * PER_W

    def chunk_body(g, carry):
        cbase = base + g * CHUNK
        pltpu.sync_copy(x_hbm.at[pl.ds(cbase, CHUNK)], xv)
        for k in range(CHUNK // L):
            xk = xv[pl.ds(k * L, L)]
            qf = jnp.clip(xk * float(BINS), 0.0, float(BINS - 1))
            qi = qf.astype(jnp.int32)
            qi = jnp.where(xk != xk, jnp.full((L,), BINS, jnp.int32), qi)
            idxv[k * L // SUB, pl.ds((k * L) % SUB, L)] = qi
        descs = [
            pltpu.async_copy(
                table_hbm.at[idxv.at[j]], rows.at[pl.ds(j * SUB, SUB)], sem
            )
            for j in range(NSUB)
        ]
        for d in descs:
            d.wait()
        pltpu.sync_copy(rows, out_hbm.at[pl.ds(cbase, CHUNK)])
        return carry

    lax.fori_loop(0, NCHUNK, chunk_body, 0)


def kernel(x, embed_weight):
    xf = x.reshape(TOTAL)
    mesh = plsc.VectorSubcoreMesh(
        core_axis_name="c", subcore_axis_name="s", num_cores=NC, num_subcores=NS
    )
    out = pl.kernel(
        _body,
        out_type=jax.ShapeDtypeStruct((TOTAL, WIDTH), jnp.float32),
        mesh=mesh,
        scratch_types=[
            pltpu.VMEM((CHUNK,), jnp.float32),
            pltpu.VMEM((NSUB, SUB), jnp.int32),
            pltpu.VMEM((CHUNK, WIDTH), jnp.float32),
            pltpu.SemaphoreType.DMA,
        ],
    )(xf, embed_weight)
    return out.reshape(x.shape[0], x.shape[1], WIDTH)


# speedup vs baseline: 3.7336x; 2.4122x over previous
import jax
import jax.numpy as jnp
from jax import lax
from jax.experimental import pallas as pl
from jax.experimental.pallas import tpu as pltpu
from jax.experimental.pallas import tpu_sc as plsc

VMIN, VMAX, BINS, WIDTH = 0.0, 1.0, 32, 128

NC, NS, L = 2, 16, 16
NW = NC * NS
TOTAL = 16384 * 50
PER_W = TOTAL // NW
CHUNK = 256
NSUB = 2
SUB = CHUNK // NSUB
NCHUNK = PER_W // CHUNK


def _body(x_hbm, table_hbm, out_hbm, xv, idxv, rows, table_v, table_sh,
          sem_g, sem_o):
    sid = lax.axis_index("s")
    wid = sid * NC + lax.axis_index("c")
    base = wid * PER_W

    @pl.when(sid == 0)
    def _stage():
        pltpu.sync_copy(table_hbm, table_v)
        pltpu.sync_copy(table_v, table_sh)

    plsc.subcore_barrier()

    def process(g, b):
        cbase = base + g * CHUNK
        pltpu.sync_copy(x_hbm.at[pl.ds(cbase, CHUNK)], xv)
        for k in range(CHUNK // L):
            xk = xv[pl.ds(k * L, L)]
            qf = jnp.clip(xk * float(BINS), 0.0, float(BINS - 1))
            qi = qf.astype(jnp.int32)
            qi = jnp.where(xk != xk, jnp.full((L,), BINS, jnp.int32), qi)
            idxv[k * L // SUB, pl.ds((k * L) % SUB, L)] = qi
        descs = [
            pltpu.async_copy(
                table_sh.at[idxv.at[j]],
                rows.at[b].at[pl.ds(j * SUB, SUB)],
                sem_g,
            )
            for j in range(NSUB)
        ]
        for d in descs:
            d.wait()
        pltpu.make_async_copy(
            rows.at[b], out_hbm.at[pl.ds(cbase, CHUNK)], sem_o
        ).start()

    process(0, 0)
    process(1, 1)

    def chunk_body(g, carry):
        b = lax.rem(g, 2)
        pltpu.make_async_copy(
            rows.at[b], out_hbm.at[pl.ds(base, CHUNK)], sem_o
        ).wait()
        process(g, b)
        return carry

    lax.fori_loop(2, NCHUNK, chunk_body, 0)
    for b in range(2):
        pltpu.make_async_copy(
            rows.at[b], out_hbm.at[pl.ds(base, CHUNK)], sem_o
        ).wait()


def kernel(x, embed_weight):
    xf = x.reshape(TOTAL)
    mesh = plsc.VectorSubcoreMesh(
        core_axis_name="c", subcore_axis_name="s", num_cores=NC, num_subcores=NS
    )
    out = pl.kernel(
        _body,
        out_type=jax.ShapeDtypeStruct((TOTAL, WIDTH), jnp.float32),
        mesh=mesh,
        scratch_types=[
            pltpu.VMEM((CHUNK,), jnp.float32),
            pltpu.VMEM((NSUB, SUB), jnp.int32),
            pltpu.VMEM((2, CHUNK, WIDTH), jnp.float32),
            pltpu.VMEM((BINS + 1, WIDTH), jnp.float32),
            pltpu.VMEM_SHARED((BINS + 1, WIDTH), jnp.float32),
            pltpu.SemaphoreType.DMA,
            pltpu.SemaphoreType.DMA,
        ],
    )(xf, embed_weight)
    return out.reshape(x.shape[0], x.shape[1], WIDTH)


# speedup vs baseline: 3.9331x; 1.0534x over previous
import jax
import jax.numpy as jnp
from jax import lax
from jax.experimental import pallas as pl
from jax.experimental.pallas import tpu as pltpu
from jax.experimental.pallas import tpu_sc as plsc

VMIN, VMAX, BINS, WIDTH = 0.0, 1.0, 32, 128

NC, NS, L = 2, 16, 16
NW = NC * NS
TOTAL = 16384 * 50
PER_W = TOTAL // NW
CHUNK = 256
NSUB = 2
SUB = CHUNK // NSUB
NCHUNK = PER_W // CHUNK


def _body(x_hbm, table_hbm, out_hbm, xv, idxv, rows, table_v, table_sh,
          sem_x0, sem_x1, sem_g0, sem_g1, sem_o0, sem_o1):
    sem_x = (sem_x0, sem_x1)
    sem_g = (sem_g0, sem_g1)
    sem_o = (sem_o0, sem_o1)
    sid = lax.axis_index("s")
    wid = sid * NC + lax.axis_index("c")
    base = wid * PER_W

    @pl.when(sid == 0)
    def _stage():
        pltpu.sync_copy(table_hbm, table_v)
        pltpu.sync_copy(table_v, table_sh)

    plsc.subcore_barrier()

    def x_copy(g, b):
        off = base + lax.min(g, NCHUNK - 1) * CHUNK
        return pltpu.make_async_copy(
            x_hbm.at[pl.ds(off, CHUNK)], xv.at[b], sem_x[b]
        )

    def quantize(b):
        for k in range(CHUNK // L):
            xk = xv[b, pl.ds(k * L, L)]
            qf = jnp.clip(xk * float(BINS), 0.0, float(BINS - 1))
            qi = qf.astype(jnp.int32)
            qi = jnp.where(xk != xk, jnp.full((L,), BINS, jnp.int32), qi)
            idxv[b, k * L // SUB, pl.ds((k * L) % SUB, L)] = qi

    def gathers(b):
        return [
            pltpu.make_async_copy(
                table_sh.at[idxv.at[b].at[j]],
                rows.at[b].at[pl.ds(j * SUB, SUB)],
                sem_g[b],
            )
            for j in range(NSUB)
        ]

    def out_copy(g, b):
        return pltpu.make_async_copy(
            rows.at[b], out_hbm.at[pl.ds(base + g * CHUNK, CHUNK)], sem_o[b]
        )

    def step(g, b, wait_out):
        x_copy(g + 1, b ^ 1).start()
        x_copy(g, b).wait()
        if wait_out:
            out_copy(g - 2, b).wait()
        quantize(b)
        for d in gathers(b):
            d.start()
        for d in gathers(b ^ 1):
            d.wait()
        out_copy(g - 1, b ^ 1).start()

    x_copy(0, 0).start()
    x_copy(0, 0).wait()
    quantize(0)
    for d in gathers(0):
        d.start()
    x_copy(1, 1).start()
    x_copy(1, 1).wait()
    quantize(1)
    for d in gathers(1):
        d.start()
    for d in gathers(0):
        d.wait()
    out_copy(0, 0).start()
    x_copy(2, 0).start()

    def chunk_body(g2, carry):
        step(2 * g2, 0, wait_out=True)
        step(2 * g2 + 1, 1, wait_out=True)
        return carry

    lax.fori_loop(1, NCHUNK // 2, chunk_body, 0)

    for d in gathers(1):
        d.wait()
    out_copy(NCHUNK - 1, 1).start()
    x_copy(NCHUNK - 1, 0).wait()
    out_copy(NCHUNK - 2, 0).wait()
    out_copy(NCHUNK - 1, 1).wait()


def kernel(x, embed_weight):
    xf = x.reshape(TOTAL)
    mesh = plsc.VectorSubcoreMesh(
        core_axis_name="c", subcore_axis_name="s", num_cores=NC, num_subcores=NS
    )
    out = pl.kernel(
        _body,
        out_type=jax.ShapeDtypeStruct((TOTAL, WIDTH), jnp.float32),
        mesh=mesh,
        scratch_types=[
            pltpu.VMEM((2, CHUNK), jnp.float32),
            pltpu.VMEM((2, NSUB, SUB), jnp.int32),
            pltpu.VMEM((2, CHUNK, WIDTH), jnp.float32),
            pltpu.VMEM((BINS + 1, WIDTH), jnp.float32),
            pltpu.VMEM_SHARED((BINS + 1, WIDTH), jnp.float32),
            pltpu.SemaphoreType.DMA,
            pltpu.SemaphoreType.DMA,
            pltpu.SemaphoreType.DMA,
            pltpu.SemaphoreType.DMA,
            pltpu.SemaphoreType.DMA,
            pltpu.SemaphoreType.DMA,
        ],
    )(xf, embed_weight)
    return out.reshape(x.shape[0], x.shape[1], WIDTH)


# speedup vs baseline: 8.2232x; 2.0908x over previous
import jax
import jax.numpy as jnp
from jax import lax
from jax.experimental import pallas as pl
from jax.experimental.pallas import tpu as pltpu
from jax.experimental.pallas import tpu_sc as plsc

VMIN, VMAX, BINS, WIDTH = 0.0, 1.0, 32, 128

NC, NS, L = 2, 16, 16
NW = NC * NS
ROWS, COLS = 16384, 50
IPW = ROWS // NW
IC = 8
NCHUNK = IPW // IC
OFFS = (0, 16, 32, 34)


def _body(x_hbm, table_hbm, out_hbm, xv, idxv, rows, table_v, table_sh,
          sem_x0, sem_x1, sem_g0, sem_g1, sem_o0, sem_o1):
    sem_x = (sem_x0, sem_x1)
    sem_g = (sem_g0, sem_g1)
    sem_o = (sem_o0, sem_o1)
    sid = lax.axis_index("s")
    wid = sid * NC + lax.axis_index("c")
    ibase = wid * IPW

    @pl.when(sid == 0)
    def _stage():
        pltpu.sync_copy(table_hbm, table_v)
        pltpu.sync_copy(table_v, table_sh)

    plsc.subcore_barrier()

    def x_copy(g, b):
        off = ibase + lax.min(g, NCHUNK - 1) * IC
        return pltpu.make_async_copy(
            x_hbm.at[pl.ds(off, IC)], xv.at[b], sem_x[b]
        )

    def quantize(b):
        for r in range(IC):
            for off in OFFS:
                xk = xv[b, r, pl.ds(off, L)]
                qf = jnp.clip(xk * float(BINS), 0.0, float(BINS - 1))
                qi = qf.astype(jnp.int32)
                qi = jnp.where(xk != xk, jnp.full((L,), BINS, jnp.int32), qi)
                idxv[b, r, pl.ds(off, L)] = qi

    def gathers(b):
        return [
            pltpu.make_async_copy(
                table_sh.at[idxv.at[b].at[r]], rows.at[b].at[r], sem_g[b]
            )
            for r in range(IC)
        ]

    def out_copy(g, b):
        return pltpu.make_async_copy(
            rows.at[b], out_hbm.at[pl.ds(ibase + g * IC, IC)], sem_o[b]
        )

    def step(g, b, wait_out):
        x_copy(g + 1, b ^ 1).start()
        x_copy(g, b).wait()
        if wait_out:
            out_copy(g - 2, b).wait()
        quantize(b)
        for d in gathers(b):
            d.start()
        for d in gathers(b ^ 1):
            d.wait()
        out_copy(g - 1, b ^ 1).start()

    x_copy(0, 0).start()
    x_copy(0, 0).wait()
    quantize(0)
    for d in gathers(0):
        d.start()
    x_copy(1, 1).start()
    x_copy(1, 1).wait()
    quantize(1)
    for d in gathers(1):
        d.start()
    for d in gathers(0):
        d.wait()
    out_copy(0, 0).start()
    x_copy(2, 0).start()

    def chunk_body(g2, carry):
        step(2 * g2, 0, wait_out=True)
        step(2 * g2 + 1, 1, wait_out=True)
        return carry

    lax.fori_loop(1, NCHUNK // 2, chunk_body, 0)

    for d in gathers(1):
        d.wait()
    out_copy(NCHUNK - 1, 1).start()
    x_copy(NCHUNK - 1, 0).wait()
    out_copy(NCHUNK - 2, 0).wait()
    out_copy(NCHUNK - 1, 1).wait()


def kernel(x, embed_weight):
    mesh = plsc.VectorSubcoreMesh(
        core_axis_name="c", subcore_axis_name="s", num_cores=NC, num_subcores=NS
    )
    out = pl.kernel(
        _body,
        out_type=jax.ShapeDtypeStruct((ROWS, COLS, WIDTH), jnp.float32),
        mesh=mesh,
        compiler_params=pltpu.CompilerParams(use_tc_tiling_on_sc=True),
        scratch_types=[
            pltpu.VMEM((2, IC, COLS), jnp.float32),
            pltpu.VMEM((2, IC, COLS), jnp.int32),
            pltpu.VMEM((2, IC, COLS, WIDTH), jnp.float32),
            pltpu.VMEM((BINS + 1, WIDTH), jnp.float32),
            pltpu.VMEM_SHARED((BINS + 1, WIDTH), jnp.float32),
            pltpu.SemaphoreType.DMA,
            pltpu.SemaphoreType.DMA,
            pltpu.SemaphoreType.DMA,
            pltpu.SemaphoreType.DMA,
            pltpu.SemaphoreType.DMA,
            pltpu.SemaphoreType.DMA,
        ],
    )(x, embed_weight)
    return out


# speedup vs baseline: 20.0973x; 2.4440x over previous
import jax
import jax.numpy as jnp
from jax import lax
from jax.experimental import pallas as pl
from jax.experimental.pallas import tpu as pltpu
from jax.experimental.pallas import tpu_sc as plsc

VMIN, VMAX, BINS, WIDTH = 0.0, 1.0, 32, 128

NC, NS, L = 2, 16, 16
NW = NC * NS
ROWS, COLS = 16384, 50
IPW = ROWS // NW
IC = 8
CN = IC * COLS
NCHUNK = IPW // IC
OFFS = (0, 16, 32, 34)
NSUB = 5
SUB = CN // NSUB


def _body(x_hbm, table_hbm, out_hbm, xv, idxg, idxd, rows, table_v, table_sh,
          sem_x0, sem_x1, sem_g0, sem_g1, sem_o0, sem_o1):
    sem_x = (sem_x0, sem_x1)
    sem_g = (sem_g0, sem_g1)
    sem_o = (sem_o0, sem_o1)
    sid = lax.axis_index("s")
    wid = sid * NC + lax.axis_index("c")
    ibase = wid * IPW

    @pl.when(sid == 0)
    def _stage():
        pltpu.sync_copy(table_hbm, table_v)
        pltpu.sync_copy(table_v, table_sh)

    plsc.subcore_barrier()

    iota = lax.iota(jnp.int32, L)

    def x_copy(g, b):
        off = ibase + lax.min(g, NCHUNK - 1) * IC
        return pltpu.make_async_copy(
            x_hbm.at[pl.ds(off, IC)], xv.at[b], sem_x[b]
        )

    def quantize(g, b):
        gi0 = ibase + g * IC
        for r in range(IC):
            for off in OFFS:
                xk = xv[b, r, pl.ds(off, L)]
                qf = jnp.clip(xk * float(BINS), 0.0, float(BINS - 1))
                qi = qf.astype(jnp.int32)
                qi = jnp.where(xk != xk, jnp.full((L,), BINS, jnp.int32), qi)
                dv = iota * ROWS + (off * ROWS + gi0 + r)
                p = r * COLS + off
                rem = p % SUB
                if rem + L <= SUB:
                    d0 = jnp.full((L,), p // SUB, jnp.int32)
                    d1 = iota + rem
                else:
                    bump = (iota >= (SUB - rem)).astype(jnp.int32)
                    d0 = bump + (p // SUB)
                    d1 = iota + rem - bump * SUB
                plsc.store_scatter(idxg.at[b], [d0, d1], qi)
                plsc.store_scatter(idxd.at[b], [d0, d1], dv)

    def gathers(b):
        return [
            pltpu.make_async_copy(
                table_sh.at[idxg.at[b].at[m]],
                rows.at[b].at[m],
                sem_g[b],
            )
            for m in range(NSUB)
        ]

    def out_scatters(b):
        return [
            pltpu.make_async_copy(
                rows.at[b].at[m],
                out_hbm.at[idxd.at[b].at[m]],
                sem_o[b],
            )
            for m in range(NSUB)
        ]

    def step(g, b, wait_out):
        x_copy(g + 1, b ^ 1).start()
        x_copy(g, b).wait()
        if wait_out:
            for d in out_scatters(b):
                d.wait()
        quantize(g, b)
        for d in gathers(b):
            d.start()
        for d in gathers(b ^ 1):
            d.wait()
        for d in out_scatters(b ^ 1):
            d.start()

    x_copy(0, 0).start()
    x_copy(0, 0).wait()
    quantize(0, 0)
    for d in gathers(0):
        d.start()
    x_copy(1, 1).start()
    x_copy(1, 1).wait()
    quantize(1, 1)
    for d in gathers(1):
        d.start()
    for d in gathers(0):
        d.wait()
    for d in out_scatters(0):
        d.start()
    x_copy(2, 0).start()

    def chunk_body(g2, carry):
        step(2 * g2, 0, wait_out=True)
        step(2 * g2 + 1, 1, wait_out=True)
        return carry

    lax.fori_loop(1, NCHUNK // 2, chunk_body, 0)

    for d in gathers(1):
        d.wait()
    for d in out_scatters(1):
        d.start()
    x_copy(NCHUNK - 1, 0).wait()
    for d in out_scatters(0):
        d.wait()
    for d in out_scatters(1):
        d.wait()


def kernel(x, embed_weight):
    mesh = plsc.VectorSubcoreMesh(
        core_axis_name="c", subcore_axis_name="s", num_cores=NC, num_subcores=NS
    )
    out = pl.kernel(
        _body,
        out_type=jax.ShapeDtypeStruct((COLS * ROWS, WIDTH), jnp.float32),
        mesh=mesh,
        compiler_params=pltpu.CompilerParams(
            use_tc_tiling_on_sc=True, needs_layout_passes=False
        ),
        scratch_types=[
            pltpu.VMEM((2, IC, COLS), jnp.float32),
            pltpu.VMEM((2, NSUB, SUB), jnp.int32),
            pltpu.VMEM((2, NSUB, SUB), jnp.int32),
            pltpu.VMEM((2, NSUB, SUB, WIDTH), jnp.float32),
            pltpu.VMEM((BINS + 1, WIDTH), jnp.float32),
            pltpu.VMEM_SHARED((BINS + 1, WIDTH), jnp.float32),
            pltpu.SemaphoreType.DMA,
            pltpu.SemaphoreType.DMA,
            pltpu.SemaphoreType.DMA,
            pltpu.SemaphoreType.DMA,
            pltpu.SemaphoreType.DMA,
            pltpu.SemaphoreType.DMA,
        ],
    )(x, embed_weight)
    return jnp.transpose(out.reshape(COLS, ROWS, WIDTH), (1, 0, 2))


# speedup vs baseline: 21.1712x; 1.0534x over previous
import jax
import jax.numpy as jnp
from jax import lax
from jax.experimental import pallas as pl
from jax.experimental.pallas import tpu as pltpu
from jax.experimental.pallas import tpu_sc as plsc

VMIN, VMAX, BINS, WIDTH = 0.0, 1.0, 32, 128

NC, NS, L = 2, 16, 16
NW = NC * NS
ROWS, COLS = 16384, 50
IPW = ROWS // NW
CHUNK = 256
HPW = IPW // CHUNK
NCHUNK = COLS * HPW
NSUB = 2
SUB = CHUNK // NSUB


def _body(xt_hbm, table_hbm, out_hbm, xv, idxg, rows, table_v, table_sh,
          sem_x0, sem_x1, sem_g0, sem_g1, sem_o0, sem_o1):
    sem_x = (sem_x0, sem_x1)
    sem_g = (sem_g0, sem_g1)
    sem_o = (sem_o0, sem_o1)
    sid = lax.axis_index("s")
    wid = sid * NC + lax.axis_index("c")
    ibase = wid * IPW

    @pl.when(sid == 0)
    def _stage():
        pltpu.sync_copy(table_hbm, table_v)
        pltpu.sync_copy(table_v, table_sh)

    plsc.subcore_barrier()

    def offs(g):
        plane = jnp.right_shift(g, 1)
        i0 = ibase + jnp.bitwise_and(g, 1) * CHUNK
        return plane, i0

    def x_copy(g, b):
        plane, i0 = offs(lax.min(g, NCHUNK - 1))
        return pltpu.make_async_copy(
            xt_hbm.at[plane, pl.ds(i0, CHUNK)], xv.at[b], sem_x[b]
        )

    def quantize(b):
        for k in range(CHUNK // L):
            xk = xv[b, pl.ds(k * L, L)]
            qf = jnp.clip(xk * float(BINS), 0.0, float(BINS - 1))
            qi = qf.astype(jnp.int32)
            qi = jnp.where(xk != xk, jnp.full((L,), BINS, jnp.int32), qi)
            idxg[b, k * L // SUB, pl.ds((k * L) % SUB, L)] = qi

    def gathers(b):
        return [
            pltpu.make_async_copy(
                table_sh.at[idxg.at[b].at[m]],
                rows.at[b].at[pl.ds(m * SUB, SUB)],
                sem_g[b],
            )
            for m in range(NSUB)
        ]

    def out_copy(g, b):
        plane, i0 = offs(g)
        return pltpu.make_async_copy(
            rows.at[b], out_hbm.at[pl.ds(plane * ROWS + i0, CHUNK)], sem_o[b]
        )

    def step(g, b, wait_out):
        x_copy(g + 1, b ^ 1).start()
        x_copy(g, b).wait()
        if wait_out:
            out_copy(g - 2, b).wait()
        quantize(b)
        for d in gathers(b):
            d.start()
        for d in gathers(b ^ 1):
            d.wait()
        out_copy(g - 1, b ^ 1).start()

    x_copy(0, 0).start()
    x_copy(0, 0).wait()
    quantize(0)
    for d in gathers(0):
        d.start()
    x_copy(1, 1).start()
    x_copy(1, 1).wait()
    quantize(1)
    for d in gathers(1):
        d.start()
    for d in gathers(0):
        d.wait()
    out_copy(0, 0).start()
    x_copy(2, 0).start()

    def chunk_body(g2, carry):
        step(2 * g2, 0, wait_out=True)
        step(2 * g2 + 1, 1, wait_out=True)
        return carry

    lax.fori_loop(1, NCHUNK // 2, chunk_body, 0)

    for d in gathers(1):
        d.wait()
    out_copy(NCHUNK - 1, 1).start()
    x_copy(NCHUNK - 1, 0).wait()
    out_copy(NCHUNK - 2, 0).wait()
    out_copy(NCHUNK - 1, 1).wait()


def kernel(x, embed_weight):
    mesh = plsc.VectorSubcoreMesh(
        core_axis_name="c", subcore_axis_name="s", num_cores=NC, num_subcores=NS
    )
    out = pl.kernel(
        _body,
        out_type=jax.ShapeDtypeStruct((COLS * ROWS, WIDTH), jnp.float32),
        mesh=mesh,
        compiler_params=pltpu.CompilerParams(
            use_tc_tiling_on_sc=True, needs_layout_passes=False
        ),
        scratch_types=[
            pltpu.VMEM((2, CHUNK), jnp.float32),
            pltpu.VMEM((2, NSUB, SUB), jnp.int32),
            pltpu.VMEM((2, CHUNK, WIDTH), jnp.float32),
            pltpu.VMEM((BINS + 1, WIDTH), jnp.float32),
            pltpu.VMEM_SHARED((BINS + 1, WIDTH), jnp.float32),
            pltpu.SemaphoreType.DMA,
            pltpu.SemaphoreType.DMA,
            pltpu.SemaphoreType.DMA,
            pltpu.SemaphoreType.DMA,
            pltpu.SemaphoreType.DMA,
            pltpu.SemaphoreType.DMA,
        ],
    )(jnp.transpose(x), embed_weight)
    return jnp.transpose(out.reshape(COLS, ROWS, WIDTH), (1, 0, 2))
